# baseline (speedup 1.0000x reference)
import jax
import jax.numpy as jnp
from jax import lax
from jax.experimental import pallas as pl
from jax.experimental.pallas import tpu as pltpu
from jax.experimental.pallas import tpu_sc as plsc

N_NODES = 100000
M_EDGES = 6400000
NUM_WORKERS = 32
CHUNK = 2048
TOTAL_CHUNKS = M_EDGES // CHUNK
BASE_CHUNKS = TOTAL_CHUNKS // NUM_WORKERS
EXTRA = TOTAL_CHUNKS - BASE_CHUNKS * NUM_WORKERS
L = 16


def _newton_sqrt(sq):
    i = plsc.bitcast(sq, jnp.int32)
    i = jnp.int32(0x5F3759DF) - (i >> 1)
    y = plsc.bitcast(i, jnp.float32)
    t = sq * jnp.float32(0.5)
    y = y * (jnp.float32(1.5) - (t * y) * y)
    y = y * (jnp.float32(1.5) - (t * y) * y)
    return sq * y


def _body(pos8_hbm, ei_hbm, w_hbm, vec_hbm,
          idx_v, rows_v, vec_v, w_v, sem):
    wid = lax.axis_index("c") * 16 + lax.axis_index("s")
    n_chunks = BASE_CHUNKS + jnp.where(wid < EXTRA, 1, 0)

    iota = lax.iota(jnp.int32, L)
    c0 = jnp.zeros((L,), jnp.int32)
    c1 = jnp.full((L,), 1, jnp.int32)
    c2 = jnp.full((L,), 2, jnp.int32)

    def chunk_body(g, carry):
        t = wid + g * NUM_WORKERS
        base = t * CHUNK
        pltpu.sync_copy(ei_hbm.at[pl.ds(2 * base, 2 * CHUNK)], idx_v)
        pltpu.async_copy(pos8_hbm.at[idx_v], rows_v, sem).wait()

        def compute(i, carry2):
            q0 = (i // 8) * 256 + (i % 8) * 16 + iota
            q1 = q0 + 128
            ep = i * L + iota
            x0 = plsc.load_gather(rows_v, [q0, c0])
            y0 = plsc.load_gather(rows_v, [q0, c1])
            z0 = plsc.load_gather(rows_v, [q0, c2])
            x1 = plsc.load_gather(rows_v, [q1, c0])
            y1 = plsc.load_gather(rows_v, [q1, c1])
            z1 = plsc.load_gather(rows_v, [q1, c2])
            dx = x0 - x1
            dy = y0 - y1
            dz = z0 - z1
            sq = dx * dx + dy * dy + dz * dz
            w = jnp.maximum(_newton_sqrt(sq), jnp.float32(1e-8))
            w_v[pl.ds(i * L, L)] = w
            plsc.store_scatter(vec_v, [ep, c0], dx)
            plsc.store_scatter(vec_v, [ep, c1], dy)
            plsc.store_scatter(vec_v, [ep, c2], dz)
            return carry2

        lax.fori_loop(0, CHUNK // L, compute, 0, unroll=False)

        pltpu.sync_copy(w_v, w_hbm.at[pl.ds(base, CHUNK)])
        pltpu.sync_copy(vec_v, vec_hbm.at[pl.ds(base, CHUNK)])
        return carry

    lax.fori_loop(0, n_chunks, chunk_body, 0, unroll=False)


_TC_K = 200
_TC_COLS = _TC_K * 128


def _detile_tc_body(ei_ref, out_ref):
    a = jnp.reshape(ei_ref[0, :], (_TC_K, 128))
    b = jnp.reshape(ei_ref[1, :], (_TC_K, 128))
    v = jnp.concatenate([a, b], axis=1)
    out_ref[...] = jnp.reshape(v, (2 * _TC_COLS,))


def _detile_tc(edge_index):
    nb = M_EDGES // _TC_COLS
    return pl.pallas_call(
        _detile_tc_body,
        grid=(nb,),
        in_specs=[pl.BlockSpec((2, _TC_COLS), lambda i: (0, i))],
        out_specs=pl.BlockSpec((2 * _TC_COLS,), lambda i: (i,)),
        out_shape=jax.ShapeDtypeStruct((2 * M_EDGES,), jnp.int32),
    )(edge_index)


def _distance_sc(pos8, ei_lin):
    mesh = plsc.VectorSubcoreMesh(core_axis_name="c", subcore_axis_name="s")
    k = pl.kernel(
        _body,
        out_type=(
            jax.ShapeDtypeStruct((M_EDGES,), jnp.float32),
            jax.ShapeDtypeStruct((M_EDGES, 3), jnp.float32),
        ),
        mesh=mesh,
        compiler_params=pltpu.CompilerParams(use_tc_tiling_on_sc=False,
                                             needs_layout_passes=False),
        scratch_types=[
            pltpu.VMEM((2 * CHUNK,), jnp.int32),
            pltpu.VMEM((2 * CHUNK, 8), jnp.float32),
            pltpu.VMEM((CHUNK, 3), jnp.float32),
            pltpu.VMEM((CHUNK,), jnp.float32),
            pltpu.SemaphoreType.DMA,
        ],
    )
    return k(pos8, ei_lin)


def kernel(pos, edge_index):
    pos8 = jnp.pad(pos, ((0, 0), (0, 5)))
    ei_lin = _detile_tc(edge_index)
    w, vec = _distance_sc(pos8, ei_lin)
    return (edge_index, w, vec)


# speedup vs baseline: 1.0375x; 1.0375x over previous
import jax
import jax.numpy as jnp
from jax import lax
from jax.experimental import pallas as pl
from jax.experimental.pallas import tpu as pltpu
from jax.experimental.pallas import tpu_sc as plsc

N_NODES = 100000
M_EDGES = 6400000
NUM_WORKERS = 32
CHUNK = 2048
TOTAL_CHUNKS = M_EDGES // CHUNK
BASE_CHUNKS = TOTAL_CHUNKS // NUM_WORKERS
EXTRA = TOTAL_CHUNKS - BASE_CHUNKS * NUM_WORKERS
L = 16


def _newton_sqrt(sq):
    i = plsc.bitcast(sq, jnp.int32)
    i = jnp.int32(0x5F3759DF) - (i >> 1)
    y = plsc.bitcast(i, jnp.float32)
    t = sq * jnp.float32(0.5)
    y = y * (jnp.float32(1.5) - (t * y) * y)
    y = y * (jnp.float32(1.5) - (t * y) * y)
    return sq * y


def _body(pos8_hbm, ei_hbm, eiout_hbm, w_hbm, vec_hbm,
          idx_v, rows_v, vec_v, w_v, sem):
    wid = lax.axis_index("c") * 16 + lax.axis_index("s")
    n_chunks = BASE_CHUNKS + jnp.where(wid < EXTRA, 1, 0)

    iota = lax.iota(jnp.int32, L)
    c0 = jnp.zeros((L,), jnp.int32)
    c1 = jnp.full((L,), 1, jnp.int32)
    c2 = jnp.full((L,), 2, jnp.int32)

    def chunk_body(g, carry):
        t = wid + g * NUM_WORKERS
        base = t * CHUNK
        pltpu.sync_copy(ei_hbm.at[pl.ds(2 * base, 2 * CHUNK)], idx_v)
        pltpu.async_copy(pos8_hbm.at[idx_v], rows_v, sem).wait()

        def compute(i, carry2):
            q0 = (i // 8) * 256 + (i % 8) * 16 + iota
            q1 = q0 + 128
            ep = i * L + iota
            x0 = plsc.load_gather(rows_v, [q0, c0])
            y0 = plsc.load_gather(rows_v, [q0, c1])
            z0 = plsc.load_gather(rows_v, [q0, c2])
            x1 = plsc.load_gather(rows_v, [q1, c0])
            y1 = plsc.load_gather(rows_v, [q1, c1])
            z1 = plsc.load_gather(rows_v, [q1, c2])
            dx = x0 - x1
            dy = y0 - y1
            dz = z0 - z1
            sq = dx * dx + dy * dy + dz * dz
            w = jnp.maximum(_newton_sqrt(sq), jnp.float32(1e-8))
            w_v[pl.ds(i * L, L)] = w
            plsc.store_scatter(vec_v, [ep, c0], dx)
            plsc.store_scatter(vec_v, [ep, c1], dy)
            plsc.store_scatter(vec_v, [ep, c2], dz)
            return carry2

        lax.fori_loop(0, CHUNK // L, compute, 0, unroll=False)

        pltpu.sync_copy(idx_v, eiout_hbm.at[pl.ds(2 * base, 2 * CHUNK)])
        pltpu.sync_copy(w_v, w_hbm.at[pl.ds(base, CHUNK)])
        pltpu.sync_copy(vec_v, vec_hbm.at[pl.ds(base, CHUNK)])
        return carry

    lax.fori_loop(0, n_chunks, chunk_body, 0, unroll=False)


def _distance_sc(pos8, ei_phys):
    mesh = plsc.VectorSubcoreMesh(core_axis_name="c", subcore_axis_name="s")
    k = pl.kernel(
        _body,
        out_type=(
            jax.ShapeDtypeStruct((2 * M_EDGES,), jnp.int32),
            jax.ShapeDtypeStruct((M_EDGES,), jnp.float32),
            jax.ShapeDtypeStruct((M_EDGES, 3), jnp.float32),
        ),
        mesh=mesh,
        compiler_params=pltpu.CompilerParams(use_tc_tiling_on_sc=False,
                                             needs_layout_passes=False),
        scratch_types=[
            pltpu.VMEM((2 * CHUNK,), jnp.int32),
            pltpu.VMEM((2 * CHUNK, 8), jnp.float32),
            pltpu.VMEM((CHUNK, 3), jnp.float32),
            pltpu.VMEM((CHUNK,), jnp.float32),
            pltpu.SemaphoreType.DMA,
        ],
    )
    return k(pos8, ei_phys)


def kernel(pos, edge_index):
    pos8 = jnp.pad(pos, ((0, 0), (0, 5)))
    nb = M_EDGES // 128
    ei_phys = jnp.reshape(
        jnp.transpose(jnp.reshape(edge_index, (2, nb, 128)), (1, 0, 2)),
        (2 * M_EDGES,))
    eiout_phys, w, vec = _distance_sc(pos8, ei_phys)
    eiout = jnp.reshape(
        jnp.transpose(jnp.reshape(eiout_phys, (nb, 2, 128)), (1, 0, 2)),
        (2, M_EDGES))
    return (eiout, w, vec)
